# baseline (speedup 1.0000x reference)
import functools

import jax
import jax.numpy as jnp
from jax import lax
from jax.experimental import pallas as pl
from jax.experimental.pallas import tpu as pltpu, tpu_sc as plsc

_CHUNK = 256


def _sc_gather_all(tail_idx, head_idx, rel_idx, entity_embedding,
                   relation_embedding, origin_relation_embedding):
    N = tail_idx.shape[0]
    B = head_idx.shape[0]
    D = entity_embedding.shape[1]

    info = plsc.get_sparse_core_info()
    NC, NS = info.num_cores, info.num_subcores
    NW = NC * NS
    n_per_w = N // NW
    b_per_w = B // NW
    n_full = n_per_w // _CHUNK
    rem = n_per_w - n_full * _CHUNK

    mesh = plsc.VectorSubcoreMesh(core_axis_name="c", subcore_axis_name="s")

    @functools.partial(
        pl.kernel,
        mesh=mesh,
        out_type=(
            jax.ShapeDtypeStruct((N, D), jnp.float32),
            jax.ShapeDtypeStruct((B, 1, D), jnp.float32),
            jax.ShapeDtypeStruct((B, 1, D), jnp.float32),
            jax.ShapeDtypeStruct((B, 1, D), jnp.float32),
        ),
        scratch_types=[
            pltpu.VMEM((n_per_w,), jnp.int32),
            pltpu.VMEM((3, _CHUNK, D), jnp.float32),
            pltpu.VMEM((rem, D), jnp.float32),
            pltpu.VMEM((b_per_w,), jnp.int32),
            pltpu.VMEM((b_per_w,), jnp.int32),
            pltpu.VMEM((b_per_w, D), jnp.float32),
            pltpu.SemaphoreType.DMA,
            pltpu.SemaphoreType.DMA,
            pltpu.SemaphoreType.DMA,
            pltpu.SemaphoreType.DMA,
            pltpu.SemaphoreType.DMA,
        ],
    )
    def k(tail_idx_hbm, head_idx_hbm, rel_idx_hbm, ent_hbm, rel_hbm, orig_hbm,
          tail_out, head_out, rel_out, orig_out,
          tidx_v, trows_v, rrows_v, hidx_v, ridx_v, srows_v,
          sem, gsem, wsem0, wsem1, wsem2):
        wsem = (wsem0, wsem1, wsem2)
        wid = lax.axis_index("s") * NC + lax.axis_index("c")
        base = wid * n_per_w
        sbase = wid * b_per_w

        pltpu.sync_copy(tail_idx_hbm.at[pl.ds(base, n_per_w)], tidx_v)
        pltpu.sync_copy(head_idx_hbm.at[pl.ds(sbase, b_per_w)], hidx_v)
        pltpu.sync_copy(rel_idx_hbm.at[pl.ds(sbase, b_per_w)], ridx_v)

        def start_gather(c, p):
            pltpu.make_async_copy(
                ent_hbm.at[tidx_v.at[pl.ds(c * _CHUNK, _CHUNK)]],
                trows_v.at[p], gsem).start()

        def wait_gather(p):
            pltpu.make_async_copy(
                ent_hbm.at[tidx_v.at[pl.ds(0, _CHUNK)]], trows_v.at[p],
                gsem).wait()

        def start_write(c, p):
            pltpu.make_async_copy(
                trows_v.at[p], tail_out.at[pl.ds(base + c * _CHUNK, _CHUNK)],
                wsem[p]).start()

        def wait_write(p):
            pltpu.make_async_copy(
                ent_hbm.at[pl.ds(0, _CHUNK)],
                tail_out.at[pl.ds(base, _CHUNK)], wsem[p]).wait()

        start_gather(0, 0)
        pltpu.async_copy(ent_hbm.at[hidx_v], srows_v, sem).wait()
        pltpu.sync_copy(srows_v, head_out.at[pl.ds(sbase, b_per_w), 0])
        pltpu.async_copy(rel_hbm.at[ridx_v], srows_v, sem).wait()
        pltpu.sync_copy(srows_v, rel_out.at[pl.ds(sbase, b_per_w), 0])
        pltpu.async_copy(orig_hbm.at[ridx_v], srows_v, sem).wait()
        pltpu.sync_copy(srows_v, orig_out.at[pl.ds(sbase, b_per_w), 0])
        roff = n_full * _CHUNK
        if rem:
            pltpu.make_async_copy(
                ent_hbm.at[tidx_v.at[pl.ds(roff, rem)]], rrows_v, sem).start()

        def step(c, p, wait_prev, fire_next):
            p1 = (p + 1) % 3
            wait_gather(p)
            if wait_prev:
                wait_write(p1)
            if fire_next:
                start_gather(c + 1, p1)
            start_write(c, p)

        step(0, 0, False, True)
        step(1, 1, False, True)
        step(2, 2, True, True)

        def body(g, _):
            for j in range(3):
                step(3 + 3 * g + j, j, True, True)
            return 0

        lax.fori_loop(0, (n_full - 5) // 3, body, 0)

        step(n_full - 2, (n_full - 2) % 3, True, True)
        step(n_full - 1, (n_full - 1) % 3, True, False)
        wait_write((n_full - 2) % 3)
        wait_write((n_full - 1) % 3)

        if rem:
            pltpu.make_async_copy(
                ent_hbm.at[tidx_v.at[pl.ds(roff, rem)]], rrows_v, sem).wait()
            pltpu.sync_copy(rrows_v, tail_out.at[pl.ds(base + roff, rem)])

    return k(tail_idx, head_idx, rel_idx, entity_embedding,
             relation_embedding, origin_relation_embedding)


def kernel(positive, negative, entity_embedding, relation_embedding,
           origin_relation_embedding):
    B, K = negative.shape[0], negative.shape[1] + 1
    D = entity_embedding.shape[1]
    tail_idx = jnp.concatenate(
        [positive[:, 2:3], negative], axis=1).T.reshape(-1)
    head_idx = positive[:, 0]
    rel_idx = positive[:, 1]
    tail, head, rel, orig = _sc_gather_all(
        tail_idx, head_idx, rel_idx, entity_embedding, relation_embedding,
        origin_relation_embedding)
    return (head, rel, tail.reshape(K, B, D).transpose(1, 0, 2), orig)
